# baseline (speedup 1.0000x reference)
import jax
import jax.numpy as jnp
from jax import lax
from jax.experimental import pallas as pl
from jax.experimental.pallas import tpu as pltpu
from jax.experimental.pallas import tpu_sc as plsc

BATCH = 16384
FIELDS = 26
NUM_FEATURES = 32
TOTAL = BATCH * FIELDS

NUM_CORES = 2
NUM_SUBCORES = 16
NW = NUM_CORES * NUM_SUBCORES

LANES = 128
BTILES = BATCH // LANES
BTILES_PER_W = BTILES // NW

CHUNK = 1024
NCHUNKS = FIELDS * (BATCH // CHUNK)
CHUNKS_PER_W = NCHUNKS // NW
NBUF = 2


def _ids_body(idsT_hbm, flat_hbm, buf_v):
    c = lax.axis_index("c")
    s = lax.axis_index("s")
    wid = s * NUM_CORES + c
    for t in range(BTILES_PER_W):
        bt = wid * BTILES_PER_W + t
        b0 = bt * LANES
        pltpu.sync_copy(idsT_hbm.at[:, pl.ds(b0, LANES)], buf_v)
        for f in range(FIELDS):
            pltpu.sync_copy(buf_v.at[f], flat_hbm.at[pl.ds(f * BATCH + b0, LANES)])


def _gather_body(flat_hbm, table_hbm, out_hbm, idx_v, rows_v, gsems, wsems):
    c = lax.axis_index("c")
    s = lax.axis_index("s")
    wid = s * NUM_CORES + c

    def stage(t, buf):
        chunk = wid * CHUNKS_PER_W + t
        f = chunk % FIELDS
        b0 = (chunk // FIELDS) * CHUNK
        pltpu.sync_copy(flat_hbm.at[pl.ds(f * BATCH + b0, CHUNK)], idx_v.at[buf])
        return pltpu.async_copy(
            table_hbm.at[idx_v.at[buf]], rows_v.at[buf], gsems[buf]
        )

    def write(t, buf):
        chunk = wid * CHUNKS_PER_W + t
        f = chunk % FIELDS
        b0 = (chunk // FIELDS) * CHUNK
        return pltpu.async_copy(
            rows_v.at[buf], out_hbm.at[pl.ds(b0, CHUNK), f], wsems[buf]
        )

    gathers = {}
    writes = {}
    for b in range(NBUF):
        gathers[b] = stage(b, b)
    for t in range(CHUNKS_PER_W):
        b = t % NBUF
        gathers.pop(b).wait()
        writes[b] = write(t, b)
        nxt = t + NBUF
        if nxt < CHUNKS_PER_W:
            writes.pop(b).wait()
            gathers[b] = stage(nxt, b)
    for w in writes.values():
        w.wait()


@jax.jit
def kernel(test_feature_ids, feature_vectors):
    mesh = plsc.VectorSubcoreMesh(core_axis_name="c", subcore_axis_name="s")
    flat_ids = pl.kernel(
        _ids_body,
        out_type=jax.ShapeDtypeStruct((TOTAL,), jnp.int32),
        mesh=mesh,
        scratch_types=[pltpu.VMEM((FIELDS, LANES), jnp.int32)],
        compiler_params=pltpu.CompilerParams(use_tc_tiling_on_sc=True),
    )(test_feature_ids.T)
    out = pl.kernel(
        _gather_body,
        out_type=jax.ShapeDtypeStruct((BATCH, FIELDS, NUM_FEATURES), jnp.float32),
        mesh=mesh,
        scratch_types=[
            pltpu.VMEM((NBUF, CHUNK), jnp.int32),
            pltpu.VMEM((NBUF, CHUNK, NUM_FEATURES), jnp.float32),
            [pltpu.SemaphoreType.DMA] * NBUF,
            [pltpu.SemaphoreType.DMA] * NBUF,
        ],
        compiler_params=pltpu.CompilerParams(use_tc_tiling_on_sc=False),
    )(flat_ids, feature_vectors)
    return out
